# baseline (speedup 1.0000x reference)
import functools

import jax
import jax.numpy as jnp
from jax import lax
from jax.experimental import pallas as pl
from jax.experimental.pallas import tpu as pltpu
from jax.experimental.pallas import tpu_sc as plsc

_N = 16777216
_L = 16
_CW = 128
_CH = 524288
_NCOPIES = _N // _CH


@functools.partial(
    pl.kernel,
    mesh=plsc.VectorSubcoreMesh(core_axis_name="c", subcore_axis_name="s"),
    out_type=(jax.ShapeDtypeStruct((_CW,), jnp.float32),
              jax.ShapeDtypeStruct((_L,), jnp.int32)),
    scratch_types=[
        pltpu.VMEM((_L,), jnp.int32),
        pltpu.VMEM((_L,), jnp.float32),
        pltpu.VMEM((_CW,), jnp.float32),
        pltpu.VMEM((_L,), jnp.int32),
    ],
)
def _sc_resolve(idx_hbm, grad_hbm, chunk_hbm, meta_hbm, iv, gv, cbuf, mbuf):
    c = lax.axis_index("c")
    s = lax.axis_index("s")

    @pl.when((c == 0) & (s == 0))
    def _():
        pltpu.sync_copy(idx_hbm, iv.at[pl.ds(0, 1)])
        pltpu.sync_copy(grad_hbm, gv.at[pl.ds(0, 1)])
        sidx = iv[...][0] % _N
        g0 = gv[...][0]
        aligned = (sidx // _CW) * _CW
        off = sidx - aligned
        lanes = lax.iota(jnp.int32, _L)
        for j in range(_CW // _L):
            cbuf[pl.ds(j * _L, _L)] = jnp.where(lanes + j * _L == off, g0, 0.0)
        mbuf[...] = jnp.full((_L,), aligned, jnp.int32)
        pltpu.sync_copy(cbuf, chunk_hbm)
        pltpu.sync_copy(mbuf, meta_hbm)


def _fill_body(out_ref, zbuf, sem):
    zbuf[...] = jnp.zeros_like(zbuf)
    copies = [
        pltpu.make_async_copy(zbuf, out_ref.at[pl.ds(k * _CH, _CH)], sem)
        for k in range(_NCOPIES)
    ]
    for cp in copies:
        cp.start()
    for cp in copies:
        cp.wait()


def _commit_body(meta_ref, zeros_ref, chunk_ref, out_ref, sem):
    del zeros_ref
    aligned = pl.multiple_of(meta_ref[0], _CW)
    cp = pltpu.make_async_copy(chunk_ref, out_ref.at[pl.ds(aligned, _CW)], sem)
    cp.start()
    cp.wait()


def kernel(grad_output, input_sizes, dim, index, out):
    del input_sizes, dim, out
    idx1 = jnp.asarray(index, jnp.int32).reshape((1,))
    grad1 = jnp.asarray(grad_output, jnp.float32).reshape((1,))

    chunk, meta = _sc_resolve(idx1, grad1)
    zeros = pl.pallas_call(
        _fill_body,
        out_specs=pl.BlockSpec(memory_space=pl.ANY),
        out_shape=jax.ShapeDtypeStruct((_N,), jnp.float32),
        scratch_shapes=[
            pltpu.VMEM((_CH,), jnp.float32),
            pltpu.SemaphoreType.DMA,
        ],
    )()
    res = pl.pallas_call(
        _commit_body,
        in_specs=[pl.BlockSpec(memory_space=pltpu.SMEM),
                  pl.BlockSpec(memory_space=pl.ANY),
                  pl.BlockSpec(memory_space=pltpu.VMEM)],
        out_specs=pl.BlockSpec(memory_space=pl.ANY),
        out_shape=jax.ShapeDtypeStruct((_N,), jnp.float32),
        input_output_aliases={1: 0},
        scratch_shapes=[pltpu.SemaphoreType.DMA],
    )(meta, zeros, chunk)
    return res


# speedup vs baseline: 1.0325x; 1.0325x over previous
import functools

import jax
import jax.numpy as jnp
from jax import lax
from jax.experimental import pallas as pl
from jax.experimental.pallas import tpu as pltpu
from jax.experimental.pallas import tpu_sc as plsc

_N = 16777216
_L = 16
_CW = 128
_CH = 524288
_NCOPIES = _N // _CH


@functools.partial(
    pl.kernel,
    mesh=plsc.VectorSubcoreMesh(core_axis_name="c", subcore_axis_name="s",
                                num_cores=1),
    out_type=(jax.ShapeDtypeStruct((_CW,), jnp.float32),
              jax.ShapeDtypeStruct((_L,), jnp.int32)),
    scratch_types=[
        pltpu.VMEM((_L,), jnp.int32),
        pltpu.VMEM((_L,), jnp.float32),
        pltpu.VMEM((_CW,), jnp.float32),
        pltpu.VMEM((_L,), jnp.int32),
    ],
)
def _sc_resolve(idx_hbm, grad_hbm, chunk_hbm, meta_hbm, iv, gv, cbuf, mbuf):
    c = lax.axis_index("c")
    s = lax.axis_index("s")

    @pl.when((c == 0) & (s == 0))
    def _():
        pltpu.sync_copy(idx_hbm, iv.at[pl.ds(0, 1)])
        pltpu.sync_copy(grad_hbm, gv.at[pl.ds(0, 1)])
        sidx = iv[...][0] % _N
        g0 = gv[...][0]
        aligned = (sidx // _CW) * _CW
        off = sidx - aligned
        lanes = lax.iota(jnp.int32, _L)
        for j in range(_CW // _L):
            cbuf[pl.ds(j * _L, _L)] = jnp.where(lanes + j * _L == off, g0, 0.0)
        mbuf[...] = jnp.full((_L,), aligned, jnp.int32)
        pltpu.sync_copy(cbuf, chunk_hbm)
        pltpu.sync_copy(mbuf, meta_hbm)


def _fill_body(out_ref, zbuf, sem):
    zbuf[...] = jnp.zeros_like(zbuf)
    copies = [
        pltpu.make_async_copy(zbuf, out_ref.at[pl.ds(k * _CH, _CH)], sem)
        for k in range(_NCOPIES)
    ]
    for cp in copies:
        cp.start()
    for cp in copies:
        cp.wait()


def _commit_body(meta_ref, zeros_ref, chunk_ref, out_ref, sem):
    del zeros_ref
    aligned = pl.multiple_of(meta_ref[0], _CW)
    cp = pltpu.make_async_copy(chunk_ref, out_ref.at[pl.ds(aligned, _CW)], sem)
    cp.start()
    cp.wait()


def kernel(grad_output, input_sizes, dim, index, out):
    del input_sizes, dim, out
    idx1 = jnp.asarray(index, jnp.int32).reshape((1,))
    grad1 = jnp.asarray(grad_output, jnp.float32).reshape((1,))

    chunk, meta = _sc_resolve(idx1, grad1)
    zeros = pl.pallas_call(
        _fill_body,
        out_specs=pl.BlockSpec(memory_space=pl.ANY),
        out_shape=jax.ShapeDtypeStruct((_N,), jnp.float32),
        scratch_shapes=[
            pltpu.VMEM((_CH,), jnp.float32),
            pltpu.SemaphoreType.DMA,
        ],
    )()
    res = pl.pallas_call(
        _commit_body,
        in_specs=[pl.BlockSpec(memory_space=pltpu.SMEM),
                  pl.BlockSpec(memory_space=pl.ANY),
                  pl.BlockSpec(memory_space=pltpu.VMEM)],
        out_specs=pl.BlockSpec(memory_space=pl.ANY),
        out_shape=jax.ShapeDtypeStruct((_N,), jnp.float32),
        input_output_aliases={1: 0},
        scratch_shapes=[pltpu.SemaphoreType.DMA],
    )(meta, zeros, chunk)
    return res


# speedup vs baseline: 1.0381x; 1.0054x over previous
import functools

import jax
import jax.numpy as jnp
from jax import lax
from jax.experimental import pallas as pl
from jax.experimental.pallas import tpu as pltpu
from jax.experimental.pallas import tpu_sc as plsc

_N = 16777216
_L = 16
_CW = 128
_CH = 131072
_NCOPIES = _N // _CH


@functools.partial(
    pl.kernel,
    mesh=plsc.VectorSubcoreMesh(core_axis_name="c", subcore_axis_name="s",
                                num_cores=1),
    out_type=(jax.ShapeDtypeStruct((_CW,), jnp.float32),
              jax.ShapeDtypeStruct((_L,), jnp.int32)),
    scratch_types=[
        pltpu.VMEM((_L,), jnp.int32),
        pltpu.VMEM((_L,), jnp.float32),
        pltpu.VMEM((_CW,), jnp.float32),
        pltpu.VMEM((_L,), jnp.int32),
    ],
)
def _sc_resolve(idx_hbm, grad_hbm, chunk_hbm, meta_hbm, iv, gv, cbuf, mbuf):
    c = lax.axis_index("c")
    s = lax.axis_index("s")

    @pl.when((c == 0) & (s == 0))
    def _():
        pltpu.sync_copy(idx_hbm, iv.at[pl.ds(0, 1)])
        pltpu.sync_copy(grad_hbm, gv.at[pl.ds(0, 1)])
        sidx = iv[...][0] % _N
        g0 = gv[...][0]
        aligned = (sidx // _CW) * _CW
        off = sidx - aligned
        lanes = lax.iota(jnp.int32, _L)
        for j in range(_CW // _L):
            cbuf[pl.ds(j * _L, _L)] = jnp.where(lanes + j * _L == off, g0, 0.0)
        mbuf[...] = jnp.full((_L,), aligned, jnp.int32)
        pltpu.sync_copy(cbuf, chunk_hbm)
        pltpu.sync_copy(mbuf, meta_hbm)


def _fill_body(out_ref, zbuf, sem):
    zbuf[...] = jnp.zeros_like(zbuf)
    copies = [
        pltpu.make_async_copy(zbuf, out_ref.at[pl.ds(k * _CH, _CH)], sem)
        for k in range(_NCOPIES)
    ]
    for cp in copies:
        cp.start()
    for cp in copies:
        cp.wait()


def _commit_body(meta_ref, zeros_ref, chunk_ref, out_ref, sem):
    del zeros_ref
    aligned = pl.multiple_of(meta_ref[0], _CW)
    cp = pltpu.make_async_copy(chunk_ref, out_ref.at[pl.ds(aligned, _CW)], sem)
    cp.start()
    cp.wait()


def kernel(grad_output, input_sizes, dim, index, out):
    del input_sizes, dim, out
    idx1 = jnp.asarray(index, jnp.int32).reshape((1,))
    grad1 = jnp.asarray(grad_output, jnp.float32).reshape((1,))

    chunk, meta = _sc_resolve(idx1, grad1)
    zeros = pl.pallas_call(
        _fill_body,
        out_specs=pl.BlockSpec(memory_space=pl.ANY),
        out_shape=jax.ShapeDtypeStruct((_N,), jnp.float32),
        scratch_shapes=[
            pltpu.VMEM((_CH,), jnp.float32),
            pltpu.SemaphoreType.DMA,
        ],
    )()
    res = pl.pallas_call(
        _commit_body,
        in_specs=[pl.BlockSpec(memory_space=pltpu.SMEM),
                  pl.BlockSpec(memory_space=pl.ANY),
                  pl.BlockSpec(memory_space=pltpu.VMEM)],
        out_specs=pl.BlockSpec(memory_space=pl.ANY),
        out_shape=jax.ShapeDtypeStruct((_N,), jnp.float32),
        input_output_aliases={1: 0},
        scratch_shapes=[pltpu.SemaphoreType.DMA],
    )(meta, zeros, chunk)
    return res
